# baseline (speedup 1.0000x reference)
import functools

import jax
import jax.numpy as jnp
from jax.experimental import pallas as pl
from jax.experimental.pallas import tpu as pltpu

_B = 16384
_TB = 2048
_K = 256
_L = 32
_BETA = 0.25


def _fused_kernel(x_ref,
                  w0, b0, w1, b1, w2, b2, w3, b3,
                  dw0, db0, dw1, db1, dw2, db2, dw3, db3,
                  cb0, cb1, cb2,
                  cs0, cs1, cs2,
                  xhat_ref, loss_ref, *, nsteps, batch):
    i = pl.program_id(0)
    x = x_ref[...]

    h = jax.nn.relu(jnp.dot(x, w0[...]) + b0[...])
    h = jax.nn.relu(jnp.dot(h, w1[...]) + b1[...])
    h = jax.nn.relu(jnp.dot(h, w2[...]) + b2[...])
    residual = jnp.dot(h, w3[...]) + b3[...]

    z_hat = jnp.zeros_like(residual)
    rq = jnp.zeros((residual.shape[0], 1), jnp.float32)
    for cb_ref, cs_ref in ((cb0, cs0), (cb1, cs1), (cb2, cs2)):
        cb = cb_ref[...]
        row_n = jnp.sum(residual * residual, axis=1, keepdims=True)
        cb_n = jnp.sum(cb * cb, axis=1)[None, :]
        cross = jax.lax.dot_general(residual, cb,
                                    (((1,), (1,)), ((), ())))
        d2 = row_n + cb_n - 2.0 * cross
        dist = jnp.sqrt(jnp.maximum(d2, 0.0))
        iota = jax.lax.broadcasted_iota(jnp.int32, (1, _K), 1)
        dmin = jnp.min(dist, axis=-1, keepdims=True)
        idx = jnp.min(jnp.where(dist == dmin, iota, _K),
                      axis=-1, keepdims=True)
        onehot = (iota == idx).astype(jnp.bfloat16)
        cs = cs_ref[...]
        cwe = (jnp.dot(onehot, cs[:_K], preferred_element_type=jnp.float32)
               + jnp.dot(onehot, cs[_K:2 * _K],
                         preferred_element_type=jnp.float32)) \
              + jnp.dot(onehot, cs[2 * _K:],
                        preferred_element_type=jnp.float32)
        residual = residual - cwe
        z_hat = z_hat + cwe
        rq = rq + jnp.sum(residual * residual, axis=1, keepdims=True)

    h = jax.nn.relu(jnp.dot(z_hat, dw0[...]) + db0[...])
    h = jax.nn.relu(jnp.dot(h, dw1[...]) + db1[...])
    h = jax.nn.relu(jnp.dot(h, dw2[...]) + db2[...])
    x_hat = jnp.dot(h, dw3[...]) + db3[...]
    xhat_ref[...] = x_hat

    diff = x_hat - x
    recon = jnp.sum(diff * diff, axis=1, keepdims=True)
    part = jnp.sum(recon + (rq + _BETA * rq))

    prev = loss_ref[...]
    acc = jnp.where(i == 0, 0.0, prev[0, 0]) + part
    loss_ref[...] = jnp.where(i == nsteps - 1, acc / batch, acc).reshape(1, 1)


def kernel(x, enc_w0, enc_b0, enc_w1, enc_b1, enc_w2, enc_b2, enc_w3, enc_b3,
           dec_w0, dec_b0, dec_w1, dec_b1, dec_w2, dec_b2, dec_w3, dec_b3,
           cb0, cb1, cb2):
    B, D = x.shape
    nsteps = B // _TB

    enc_ws = [w.T for w in (enc_w0, enc_w1, enc_w2, enc_w3)]
    enc_bs = [b[None, :] for b in (enc_b0, enc_b1, enc_b2, enc_b3)]
    dec_ws = [w.T for w in (dec_w0, dec_w1, dec_w2, dec_w3)]
    dec_bs = [b[None, :] for b in (dec_b0, dec_b1, dec_b2, dec_b3)]

    def rep(a):
        return pl.BlockSpec(a.shape, lambda i: (0, 0))

    in_specs = [pl.BlockSpec((_TB, D), lambda i: (i, 0))]
    operands = [x]
    for w, b in zip(enc_ws, enc_bs):
        in_specs += [rep(w), rep(b)]
        operands += [w, b]
    for w, b in zip(dec_ws, dec_bs):
        in_specs += [rep(w), rep(b)]
        operands += [w, b]
    for cb in (cb0, cb1, cb2):
        in_specs.append(rep(cb))
        operands.append(cb)
    def trunc16(v):
        return jax.lax.bitcast_convert_type(
            jax.lax.bitcast_convert_type(v, jnp.int32) & jnp.int32(-65536),
            jnp.float32)

    for cb in (cb0, cb1, cb2):
        c0 = trunc16(cb)
        r1 = cb - c0
        c1 = trunc16(r1)
        c2 = r1 - c1
        cs = jnp.concatenate([c0, c1, c2], axis=0).astype(jnp.bfloat16)
        in_specs.append(rep(cs))
        operands.append(cs)

    out_specs = (pl.BlockSpec((_TB, D), lambda i: (i, 0)),
                 pl.BlockSpec((1, 1), lambda i: (0, 0)))
    out_shape = (jax.ShapeDtypeStruct((B, D), jnp.float32),
                 jax.ShapeDtypeStruct((1, 1), jnp.float32))

    x_hat, loss = pl.pallas_call(
        functools.partial(_fused_kernel, nsteps=nsteps, batch=float(B)),
        grid=(nsteps,),
        in_specs=in_specs,
        out_specs=out_specs,
        out_shape=out_shape,
        compiler_params=pltpu.CompilerParams(
            dimension_semantics=("arbitrary",)),
    )(*operands)
    return x_hat, loss[0, 0]


# speedup vs baseline: 1.0233x; 1.0233x over previous
import functools

import jax
import jax.numpy as jnp
from jax.experimental import pallas as pl
from jax.experimental.pallas import tpu as pltpu

_B = 16384
_TB = 2048
_K = 256
_L = 32
_BETA = 0.25


def _fused_kernel(x_ref,
                  w0, b0, w1, b1, w2, b2, w3, b3,
                  dw0, db0, dw1, db1, dw2, db2, dw3, db3,
                  cb0, cb1, cb2,
                  cm0, cm1, cm2,
                  cs0, cs1, cs2,
                  xhat_ref, loss_ref, *, nsteps, batch):
    i = pl.program_id(0)
    x = x_ref[...]

    h = jax.nn.relu(jnp.dot(x, w0[...]) + b0[...])
    h = jax.nn.relu(jnp.dot(h, w1[...]) + b1[...])
    h = jax.nn.relu(jnp.dot(h, w2[...]) + b2[...])
    residual = jnp.dot(h, w3[...]) + b3[...]

    z_hat = jnp.zeros_like(residual)
    rq = jnp.zeros((residual.shape[0], 1), jnp.float32)
    iota = jax.lax.broadcasted_iota(jnp.int32, (1, _K), 1)
    iota3 = jax.lax.broadcasted_iota(jnp.int32, (1, 3 * _K), 1) & (_K - 1)
    for cb_ref, cbm2_ref, cs_ref in ((cb0, cm0, cs0), (cb1, cm1, cs1),
                                     (cb2, cm2, cs2)):
        cb = cb_ref[...]
        row_n = jnp.sum(residual * residual, axis=1, keepdims=True)
        cb_n = jnp.sum(cb * cb, axis=1)[None, :]
        cross2 = jax.lax.dot_general(residual, cbm2_ref[...],
                                     (((1,), (1,)), ((), ())))
        d2 = (row_n + cb_n) + cross2
        dist = jnp.sqrt(jnp.maximum(d2, 0.0))
        dmin = jnp.min(dist, axis=-1, keepdims=True)
        idx = jnp.min(jnp.where(dist == dmin, iota, _K),
                      axis=-1, keepdims=True)
        onehot3 = (iota3 == idx).astype(jnp.bfloat16)
        cwe = jnp.dot(onehot3, cs_ref[...],
                      preferred_element_type=jnp.float32)
        residual = residual - cwe
        z_hat = z_hat + cwe
        rq = rq + jnp.sum(residual * residual, axis=1, keepdims=True)

    h = jax.nn.relu(jnp.dot(z_hat, dw0[...]) + db0[...])
    h = jax.nn.relu(jnp.dot(h, dw1[...]) + db1[...])
    h = jax.nn.relu(jnp.dot(h, dw2[...]) + db2[...])
    x_hat = jnp.dot(h, dw3[...]) + db3[...]
    xhat_ref[...] = x_hat

    diff = x_hat - x
    recon = jnp.sum(diff * diff, axis=1, keepdims=True)
    part = jnp.sum(recon + (rq + _BETA * rq))

    prev = loss_ref[...]
    acc = jnp.where(i == 0, 0.0, prev[0, 0]) + part
    loss_ref[...] = jnp.where(i == nsteps - 1, acc / batch, acc).reshape(1, 1)


def kernel(x, enc_w0, enc_b0, enc_w1, enc_b1, enc_w2, enc_b2, enc_w3, enc_b3,
           dec_w0, dec_b0, dec_w1, dec_b1, dec_w2, dec_b2, dec_w3, dec_b3,
           cb0, cb1, cb2):
    B, D = x.shape
    nsteps = B // _TB

    enc_ws = [w.T for w in (enc_w0, enc_w1, enc_w2, enc_w3)]
    enc_bs = [b[None, :] for b in (enc_b0, enc_b1, enc_b2, enc_b3)]
    dec_ws = [w.T for w in (dec_w0, dec_w1, dec_w2, dec_w3)]
    dec_bs = [b[None, :] for b in (dec_b0, dec_b1, dec_b2, dec_b3)]

    def rep(a):
        return pl.BlockSpec(a.shape, lambda i: (0, 0))

    in_specs = [pl.BlockSpec((_TB, D), lambda i: (i, 0))]
    operands = [x]
    for w, b in zip(enc_ws, enc_bs):
        in_specs += [rep(w), rep(b)]
        operands += [w, b]
    for w, b in zip(dec_ws, dec_bs):
        in_specs += [rep(w), rep(b)]
        operands += [w, b]
    for cb in (cb0, cb1, cb2):
        in_specs.append(rep(cb))
        operands.append(cb)
    for cb in (cb0, cb1, cb2):
        cbm2 = cb * jnp.float32(-2.0)
        in_specs.append(rep(cbm2))
        operands.append(cbm2)
    def trunc16(v):
        return jax.lax.bitcast_convert_type(
            jax.lax.bitcast_convert_type(v, jnp.int32) & jnp.int32(-65536),
            jnp.float32)

    for cb in (cb0, cb1, cb2):
        c0 = trunc16(cb)
        r1 = cb - c0
        c1 = trunc16(r1)
        c2 = r1 - c1
        cs = jnp.concatenate([c0, c1, c2], axis=0).astype(jnp.bfloat16)
        in_specs.append(rep(cs))
        operands.append(cs)

    out_specs = (pl.BlockSpec((_TB, D), lambda i: (i, 0)),
                 pl.BlockSpec((1, 1), lambda i: (0, 0)))
    out_shape = (jax.ShapeDtypeStruct((B, D), jnp.float32),
                 jax.ShapeDtypeStruct((1, 1), jnp.float32))

    x_hat, loss = pl.pallas_call(
        functools.partial(_fused_kernel, nsteps=nsteps, batch=float(B)),
        grid=(nsteps,),
        in_specs=in_specs,
        out_specs=out_specs,
        out_shape=out_shape,
        compiler_params=pltpu.CompilerParams(
            dimension_semantics=("arbitrary",)),
    )(*operands)
    return x_hat, loss[0, 0]
